# Initial kernel scaffold; baseline (speedup 1.0000x reference)
import jax
import jax.numpy as jnp
from jax.experimental import pallas as pl


def kernel(positions, idx_i, idx_j, offsets):
    raise NotImplementedError("write your pallas kernel here")


# speedup vs baseline: 3.5282x; 3.5282x over previous
import functools

import jax
import jax.numpy as jnp
from jax import lax
from jax.experimental import pallas as pl
from jax.experimental.pallas import tpu as pltpu
from jax.experimental.pallas import tpu_sc as plsc

N_NODES = 100000
N_EDGES = 3200000

LANES = 16
IDXW = 128
CHR = 8
CHUNK = CHR * IDXW
ROWS = N_EDGES // IDXW
NCHUNKS = ROWS // CHR
NWORKERS = 32
NSUB = 16
NR_PER_SUB = N_NODES // NSUB

_RSQRT_MAGIC = 0x5F3759DF


def _rsqrt(s):
    bits = plsc.bitcast(s, jnp.int32)
    y = plsc.bitcast(_RSQRT_MAGIC - (bits >> 1), jnp.float32)
    half_s = 0.5 * s
    for _ in range(3):
        y = y * (1.5 - half_s * y * y)
    return y


def _sc_body(pos_hbm, idxi_hbm, idxj_hbm, offs_hbm, zeros_hbm,
             dist_hbm, gradp_hbm,
             idxi_v, idxj_v, offs_v, posi_v, posj_v, uj_v, ui_v, dist_v,
             gsem, grad_sh):
    cid = lax.axis_index("c")
    sid = lax.axis_index("s")
    wid = sid * 2 + cid

    rows0 = sid * NR_PER_SUB
    pltpu.sync_copy(zeros_hbm.at[pl.ds(rows0, NR_PER_SUB)],
                    grad_sh.at[pl.ds(rows0, NR_PER_SUB)])
    plsc.subcore_barrier()

    iota16 = lax.iota(jnp.int32, LANES)
    c0 = jnp.zeros((LANES,), jnp.int32)
    c1 = jnp.full((LANES,), 1, jnp.int32)
    c2 = jnp.full((LANES,), 2, jnp.int32)

    nloc = (NCHUNKS - wid + NWORKERS - 1) // NWORKERS

    def chunk_body(t, carry):
        g = wid + t * NWORKERS
        r0 = g * CHR
        e0 = g * CHUNK
        pltpu.sync_copy(idxi_hbm.at[pl.ds(r0, CHR)], idxi_v)
        pltpu.sync_copy(idxj_hbm.at[pl.ds(r0, CHR)], idxj_v)
        pltpu.sync_copy(offs_hbm.at[pl.ds(e0, CHUNK)], offs_v)
        cps = []
        for j in range(CHR):
            cps.append(pltpu.async_copy(
                pos_hbm.at[idxi_v.at[j]], posi_v.at[pl.ds(j * IDXW, IDXW)], gsem))
            cps.append(pltpu.async_copy(
                pos_hbm.at[idxj_v.at[j]], posj_v.at[pl.ds(j * IDXW, IDXW)], gsem))
        for cp in cps:
            cp.wait()

        def grp(i, c):
            base = i * LANES
            ridx = base + iota16
            pxi = plsc.load_gather(posi_v, [ridx, c0])
            pyi = plsc.load_gather(posi_v, [ridx, c1])
            pzi = plsc.load_gather(posi_v, [ridx, c2])
            pxj = plsc.load_gather(posj_v, [ridx, c0])
            pyj = plsc.load_gather(posj_v, [ridx, c1])
            pzj = plsc.load_gather(posj_v, [ridx, c2])
            ox = plsc.load_gather(offs_v, [ridx, c0])
            oy = plsc.load_gather(offs_v, [ridx, c1])
            oz = plsc.load_gather(offs_v, [ridx, c2])
            rx = pxj - pxi + ox
            ry = pyj - pyi + oy
            rz = pzj - pzi + oz
            s = rx * rx + ry * ry + rz * rz
            y = _rsqrt(s)
            dist_v[pl.ds(base, LANES)] = s * y
            ux = rx * y
            uy = ry * y
            uz = rz * y
            plsc.store_scatter(uj_v, [ridx, c0], ux)
            plsc.store_scatter(uj_v, [ridx, c1], uy)
            plsc.store_scatter(uj_v, [ridx, c2], uz)
            plsc.store_scatter(ui_v, [ridx, c0], -ux)
            plsc.store_scatter(ui_v, [ridx, c1], -uy)
            plsc.store_scatter(ui_v, [ridx, c2], -uz)
            return c

        lax.fori_loop(0, CHUNK // LANES, grp, 0)
        pltpu.sync_copy(dist_v, dist_hbm.at[pl.ds(e0, CHUNK)])
        for j in range(CHR):
            pltpu.sync_copy(uj_v.at[pl.ds(j * IDXW, IDXW)],
                            grad_sh.at[idxj_v.at[j]], add=True)
            pltpu.sync_copy(ui_v.at[pl.ds(j * IDXW, IDXW)],
                            grad_sh.at[idxi_v.at[j]], add=True)
        return carry

    lax.fori_loop(0, nloc, chunk_body, 0)
    plsc.subcore_barrier()
    pltpu.sync_copy(grad_sh.at[pl.ds(rows0, NR_PER_SUB)],
                    gradp_hbm.at[cid, pl.ds(rows0, NR_PER_SUB)])


@jax.jit
def _sc_call(pos4, idxi2, idxj2, offsets, zeros):
    mesh = plsc.VectorSubcoreMesh(core_axis_name="c", subcore_axis_name="s")
    f = pl.kernel(
        _sc_body,
        out_type=(
            jax.ShapeDtypeStruct((N_EDGES,), jnp.float32),
            jax.ShapeDtypeStruct((2, N_NODES, 8), jnp.float32),
        ),
        mesh=mesh,
        compiler_params=pltpu.CompilerParams(use_tc_tiling_on_sc=False,
                                              needs_layout_passes=False),
        scratch_types=[
            pltpu.VMEM((CHR, IDXW), jnp.int32),
            pltpu.VMEM((CHR, IDXW), jnp.int32),
            pltpu.VMEM((CHUNK, 3), jnp.float32),
            pltpu.VMEM((CHUNK, 8), jnp.float32),
            pltpu.VMEM((CHUNK, 8), jnp.float32),
            pltpu.VMEM((CHUNK, 8), jnp.float32),
            pltpu.VMEM((CHUNK, 8), jnp.float32),
            pltpu.VMEM((CHUNK,), jnp.float32),
            pltpu.SemaphoreType.DMA,
            pltpu.VMEM_SHARED((N_NODES, 8), jnp.float32),
        ],
    )
    return f(pos4, idxi2, idxj2, offsets, zeros)


def kernel(positions, idx_i, idx_j, offsets):
    pos4 = jnp.pad(positions, ((0, 0), (0, 5)))
    ii = idx_i.astype(jnp.int32).reshape(ROWS, IDXW)
    ij = idx_j.astype(jnp.int32).reshape(ROWS, IDXW)
    zeros = jnp.zeros((N_NODES, 8), jnp.float32)
    dist, gradp = _sc_call(pos4, ii, ij, offsets, zeros)
    grad = (gradp[0] + gradp[1])[:, :3]
    return (dist, grad)


# speedup vs baseline: 3.6424x; 1.0324x over previous
import jax
import jax.numpy as jnp
from jax import lax
from jax.experimental import pallas as pl
from jax.experimental.pallas import tpu as pltpu
from jax.experimental.pallas import tpu_sc as plsc

N_NODES = 100000
N_EDGES = 3200000

LANES = 16
D = 8
C = 2000
SUB = 400
NSUBC = C // SUB
E_PER_TILE = N_EDGES // 32
NCH = E_PER_TILE // C
UNROLL = 10
IDX_RING = 5
NSUB = 16
NR_PER_SUB = 6256
N_PAD = NSUB * NR_PER_SUB


def _rsqrt(s):
    bits = plsc.bitcast(s, jnp.int32)
    y = plsc.bitcast(0x5F3759DF - (bits >> 1), jnp.float32)
    half_s = 0.5 * s
    for _ in range(3):
        y = y * (1.5 - half_s * y * y)
    return y


def _sc_body(pos_hbm, idxi_hbm, idxj_hbm, offs_hbm, zeros_hbm,
             dist_hbm, gradp_hbm, *scratch):
    idxi_r = scratch[0:IDX_RING]
    idxj_r = scratch[IDX_RING:2 * IDX_RING]
    k = 2 * IDX_RING
    offs_v = scratch[k]
    posi_s = scratch[k + 1:k + 3]
    posj_s = scratch[k + 3:k + 5]
    u_v = scratch[k + 5:k + 11]
    dist_r = scratch[k + 11:k + 13]
    gsem = scratch[k + 13:k + 15]
    osem = scratch[k + 15]
    ssem = scratch[k + 16]
    g_sh = scratch[k + 17:k + 20]

    cid = lax.axis_index("c")
    sid = lax.axis_index("s")
    wid = sid * 2 + cid
    span = wid * E_PER_TILE

    rows0 = sid * NR_PER_SUB
    for g in g_sh:
        pltpu.sync_copy(zeros_hbm, g.at[pl.ds(rows0, NR_PER_SUB)])
    plsc.subcore_barrier()

    iota16 = lax.iota(jnp.int32, LANES)
    cols = [jnp.full((LANES,), kk, jnp.int32) for kk in range(3)]

    def sub_descs(ring, s, spar):
        ii = idxi_r[ring].at[pl.ds(s * SUB, SUB)]
        jj = idxj_r[ring].at[pl.ds(s * SUB, SUB)]
        return (pltpu.make_async_copy(pos_hbm.at[ii], posi_s[spar],
                                      gsem[spar]),
                pltpu.make_async_copy(pos_hbm.at[jj], posj_s[spar],
                                      gsem[spar]))

    def fire_sub(ring, s, spar):
        a, b = sub_descs(ring, s, spar)
        a.start()
        b.start()

    def wait_sub(ring, s, spar):
        a, b = sub_descs(ring, s, spar)
        a.wait()
        b.wait()

    def scat_descs(ring, par, e0):
        descs = []
        for comp in range(3):
            descs.append(pltpu.make_async_copy(
                u_v[comp], g_sh[comp].at[idxj_r[ring]], ssem))
            descs.append(pltpu.make_async_copy(
                u_v[3 + comp], g_sh[comp].at[idxi_r[ring]], ssem))
        descs.append(pltpu.make_async_copy(
            dist_r[par], dist_hbm.at[pl.ds(e0, C)], ssem))
        return descs

    pltpu.sync_copy(idxi_hbm.at[pl.ds(span, C)], idxi_r[0])
    pltpu.sync_copy(idxj_hbm.at[pl.ds(span, C)], idxj_r[0])
    pltpu.sync_copy(offs_hbm.at[pl.ds(span, C)], offs_v)
    fire_sub(0, 0, 0)

    def compute_sub(s, spar, par):
        posi_v, posj_v = posi_s[spar], posj_s[spar]
        dist_v = dist_r[par]
        base = s * SUB

        def grp(i, carry):
            lridx = i * LANES + iota16
            cbase = base + i * LANES
            cridx = cbase + iota16
            pxi = plsc.load_gather(posi_v, [lridx, cols[0]])
            pyi = plsc.load_gather(posi_v, [lridx, cols[1]])
            pzi = plsc.load_gather(posi_v, [lridx, cols[2]])
            pxj = plsc.load_gather(posj_v, [lridx, cols[0]])
            pyj = plsc.load_gather(posj_v, [lridx, cols[1]])
            pzj = plsc.load_gather(posj_v, [lridx, cols[2]])
            ox = plsc.load_gather(offs_v, [cridx, cols[0]])
            oy = plsc.load_gather(offs_v, [cridx, cols[1]])
            oz = plsc.load_gather(offs_v, [cridx, cols[2]])
            rx = pxj - pxi + ox
            ry = pyj - pyi + oy
            rz = pzj - pzi + oz
            s2 = rx * rx + ry * ry + rz * rz
            y = _rsqrt(s2)
            dist_v[pl.ds(cbase, LANES)] = s2 * y
            ux, uy, uz = rx * y, ry * y, rz * y
            u_v[0][pl.ds(cbase, LANES)] = ux
            u_v[1][pl.ds(cbase, LANES)] = uy
            u_v[2][pl.ds(cbase, LANES)] = uz
            u_v[3][pl.ds(cbase, LANES)] = -ux
            u_v[4][pl.ds(cbase, LANES)] = -uy
            u_v[5][pl.ds(cbase, LANES)] = -uz
            return carry

        lax.fori_loop(0, SUB // LANES, grp, 0)

    def slot(t, u):
        ring = u % IDX_RING
        ring_n = (u + 1) % IDX_RING
        ring_p = (u - 1) % IDX_RING
        par = u % 2
        par_p = (u - 1) % 2
        e0 = span + t * C

        @pl.when(t > 0)
        def _():
            pltpu.make_async_copy(offs_hbm.at[pl.ds(e0, C)],
                                  offs_v, osem).wait()

        wait_sub(ring, 0, 0)

        for s in range(NSUBC):
            if s + 1 < NSUBC:
                fire_sub(ring, s + 1, (s + 1) % 2)
            if s > 0:
                wait_sub(ring, s, s % 2)
            compute_sub(s, s % 2, par)

        for comp in range(3):
            pltpu.sync_copy(u_v[comp], g_sh[comp].at[idxj_r[ring]], add=True)
            pltpu.sync_copy(u_v[3 + comp], g_sh[comp].at[idxi_r[ring]],
                            add=True)
        pltpu.sync_copy(dist_r[par], dist_hbm.at[pl.ds(e0, C)])

        @pl.when(t + 1 < NCH)
        def _():
            en = e0 + C
            pltpu.sync_copy(idxi_hbm.at[pl.ds(en, C)], idxi_r[ring_n])
            pltpu.sync_copy(idxj_hbm.at[pl.ds(en, C)], idxj_r[ring_n])
            pltpu.async_copy(offs_hbm.at[pl.ds(en, C)], offs_v, osem)
            fire_sub(ring_n, 0, 0)

    def outer(kk, carry):
        for u in range(UNROLL):
            slot(kk * UNROLL + u, u)
        return carry

    lax.fori_loop(0, NCH // UNROLL, outer, 0)

    plsc.subcore_barrier()
    for comp in range(3):
        pltpu.sync_copy(g_sh[comp].at[pl.ds(rows0, NR_PER_SUB)],
                        gradp_hbm.at[cid, comp, pl.ds(rows0, NR_PER_SUB)])


@jax.jit
def _sc_call(pos8, idx_i, idx_j, offsets, zeros):
    mesh = plsc.VectorSubcoreMesh(core_axis_name="c", subcore_axis_name="s")
    scratch = (
        [pltpu.VMEM((C,), jnp.int32) for _ in range(2 * IDX_RING)]
        + [pltpu.VMEM((C, 3), jnp.float32)]
        + [pltpu.VMEM((SUB, D), jnp.float32) for _ in range(4)]
        + [pltpu.VMEM((C,), jnp.float32) for _ in range(6)]
        + [pltpu.VMEM((C,), jnp.float32) for _ in range(2)]
        + [pltpu.SemaphoreType.DMA for _ in range(4)]
        + [pltpu.VMEM_SHARED((N_PAD,), jnp.float32) for _ in range(3)]
    )
    f = pl.kernel(
        _sc_body,
        out_type=(
            jax.ShapeDtypeStruct((N_EDGES,), jnp.float32),
            jax.ShapeDtypeStruct((2, 3, N_PAD), jnp.float32),
        ),
        mesh=mesh,
        compiler_params=pltpu.CompilerParams(use_tc_tiling_on_sc=False,
                                             needs_layout_passes=False),
        scratch_types=scratch,
    )
    return f(pos8, idx_i, idx_j, offsets, zeros)


def kernel(positions, idx_i, idx_j, offsets):
    pos8 = jnp.pad(positions, ((0, 0), (0, D - 3)))
    ii = idx_i.astype(jnp.int32)
    ij = idx_j.astype(jnp.int32)
    zeros = jnp.zeros((NR_PER_SUB,), jnp.float32)
    dist, gradp = _sc_call(pos8, ii, ij, offsets, zeros)
    grad = (gradp[0] + gradp[1])[:, :N_NODES].T
    return (dist, grad)


# speedup vs baseline: 3.9385x; 1.0813x over previous
import jax
import jax.numpy as jnp
from jax import lax
from jax.experimental import pallas as pl
from jax.experimental.pallas import tpu as pltpu
from jax.experimental.pallas import tpu_sc as plsc

N_NODES = 100000
N_EDGES = 3200000

LANES = 16
D = 8
C = 2000
SUB = 400
NSUBC = C // SUB
E_PER_TILE = N_EDGES // 32
NCH = E_PER_TILE // C
UNROLL = 10
IDX_RING = 5
NSUB = 16
NR_PER_SUB = 6256
N_PAD = NSUB * NR_PER_SUB


def _rsqrt(s):
    bits = plsc.bitcast(s, jnp.int32)
    y = plsc.bitcast(0x5F3759DF - (bits >> 1), jnp.float32)
    half_s = 0.5 * s
    for _ in range(3):
        y = y * (1.5 - half_s * y * y)
    return y


def _sc_body(pos_hbm, idxi_hbm, idxj_hbm, offs_hbm, zeros_hbm,
             dist_hbm, gradp_hbm, *scratch):
    idxi_r = scratch[0:IDX_RING]
    idxj_r = scratch[IDX_RING:2 * IDX_RING]
    k = 2 * IDX_RING
    offs_v = scratch[k]
    posi_s = scratch[k + 1:k + 3]
    posj_s = scratch[k + 3:k + 5]
    u_v = scratch[k + 5:k + 11]
    dist_r = scratch[k + 11:k + 13]
    gsem = scratch[k + 13:k + 15]
    osem = scratch[k + 15]
    ssem = scratch[k + 16]
    g_sh = scratch[k + 17:k + 20]

    cid = lax.axis_index("c")
    sid = lax.axis_index("s")
    wid = sid * 2 + cid
    span = wid * E_PER_TILE

    rows0 = sid * NR_PER_SUB
    for g in g_sh:
        pltpu.sync_copy(zeros_hbm, g.at[pl.ds(rows0, NR_PER_SUB)])
    plsc.subcore_barrier()

    iota16 = lax.iota(jnp.int32, LANES)
    iota3 = iota16 * 3
    cols = [jnp.full((LANES,), kk, jnp.int32) for kk in range(3)]

    def sub_descs(ring, s, spar):
        ii = idxi_r[ring].at[pl.ds(s * SUB, SUB)]
        jj = idxj_r[ring].at[pl.ds(s * SUB, SUB)]
        return (pltpu.make_async_copy(pos_hbm.at[ii], posi_s[spar],
                                      gsem[spar]),
                pltpu.make_async_copy(pos_hbm.at[jj], posj_s[spar],
                                      gsem[spar]))

    def fire_sub(ring, s, spar):
        a, b = sub_descs(ring, s, spar)
        a.start()
        b.start()

    def wait_sub(ring, s, spar):
        a, b = sub_descs(ring, s, spar)
        a.wait()
        b.wait()

    def scat_descs(ring, par, e0):
        descs = []
        for comp in range(3):
            descs.append(pltpu.make_async_copy(
                u_v[comp], g_sh[comp].at[idxj_r[ring]], ssem))
            descs.append(pltpu.make_async_copy(
                u_v[3 + comp], g_sh[comp].at[idxi_r[ring]], ssem))
        descs.append(pltpu.make_async_copy(
            dist_r[par], dist_hbm.at[pl.ds(e0, C)], ssem))
        return descs

    pltpu.sync_copy(idxi_hbm.at[pl.ds(span, C)], idxi_r[0])
    pltpu.sync_copy(idxj_hbm.at[pl.ds(span, C)], idxj_r[0])
    pltpu.sync_copy(offs_hbm.at[pl.ds(3 * span, 3 * C)], offs_v)
    fire_sub(0, 0, 0)

    def compute_sub(s, spar, par):
        posi_v, posj_v = posi_s[spar], posj_s[spar]
        dist_v = dist_r[par]
        base = s * SUB

        def grp(i, carry):
            lridx = i * LANES + iota16
            cbase = base + i * LANES
            cridx = cbase + iota16
            pxi = plsc.load_gather(posi_v, [lridx, cols[0]])
            pyi = plsc.load_gather(posi_v, [lridx, cols[1]])
            pzi = plsc.load_gather(posi_v, [lridx, cols[2]])
            pxj = plsc.load_gather(posj_v, [lridx, cols[0]])
            pyj = plsc.load_gather(posj_v, [lridx, cols[1]])
            pzj = plsc.load_gather(posj_v, [lridx, cols[2]])
            obase = 3 * cbase + iota3
            ox = plsc.load_gather(offs_v, [obase])
            oy = plsc.load_gather(offs_v, [obase + 1])
            oz = plsc.load_gather(offs_v, [obase + 2])
            rx = pxj - pxi + ox
            ry = pyj - pyi + oy
            rz = pzj - pzi + oz
            s2 = rx * rx + ry * ry + rz * rz
            y = _rsqrt(s2)
            dist_v[pl.ds(cbase, LANES)] = s2 * y
            ux, uy, uz = rx * y, ry * y, rz * y
            u_v[0][pl.ds(cbase, LANES)] = ux
            u_v[1][pl.ds(cbase, LANES)] = uy
            u_v[2][pl.ds(cbase, LANES)] = uz
            u_v[3][pl.ds(cbase, LANES)] = -ux
            u_v[4][pl.ds(cbase, LANES)] = -uy
            u_v[5][pl.ds(cbase, LANES)] = -uz
            return carry

        lax.fori_loop(0, SUB // LANES, grp, 0)

    def slot(t, u):
        ring = u % IDX_RING
        ring_n = (u + 1) % IDX_RING
        ring_p = (u - 1) % IDX_RING
        par = u % 2
        par_p = (u - 1) % 2
        e0 = span + t * C

        @pl.when(t > 0)
        def _():
            pltpu.make_async_copy(offs_hbm.at[pl.ds(3 * e0, 3 * C)],
                                  offs_v, osem).wait()

        wait_sub(ring, 0, 0)

        for s in range(NSUBC):
            if s + 1 < NSUBC:
                fire_sub(ring, s + 1, (s + 1) % 2)
            if s > 0:
                wait_sub(ring, s, s % 2)
            compute_sub(s, s % 2, par)

        for comp in range(3):
            pltpu.sync_copy(u_v[comp], g_sh[comp].at[idxj_r[ring]], add=True)
            pltpu.sync_copy(u_v[3 + comp], g_sh[comp].at[idxi_r[ring]],
                            add=True)
        pltpu.sync_copy(dist_r[par], dist_hbm.at[pl.ds(e0, C)])

        @pl.when(t + 1 < NCH)
        def _():
            en = e0 + C
            pltpu.sync_copy(idxi_hbm.at[pl.ds(en, C)], idxi_r[ring_n])
            pltpu.sync_copy(idxj_hbm.at[pl.ds(en, C)], idxj_r[ring_n])
            pltpu.async_copy(offs_hbm.at[pl.ds(3 * en, 3 * C)], offs_v, osem)
            fire_sub(ring_n, 0, 0)

    def outer(kk, carry):
        for u in range(UNROLL):
            slot(kk * UNROLL + u, u)
        return carry

    lax.fori_loop(0, NCH // UNROLL, outer, 0)

    plsc.subcore_barrier()
    for comp in range(3):
        off = (cid * 3 + comp) * N_PAD + rows0
        pltpu.sync_copy(g_sh[comp].at[pl.ds(rows0, NR_PER_SUB)],
                        gradp_hbm.at[pl.ds(off, NR_PER_SUB)])


@jax.jit
def _sc_call(pos8, idx_i, idx_j, offsets, zeros):
    mesh = plsc.VectorSubcoreMesh(core_axis_name="c", subcore_axis_name="s")
    scratch = (
        [pltpu.VMEM((C,), jnp.int32) for _ in range(2 * IDX_RING)]
        + [pltpu.VMEM((3 * C,), jnp.float32)]
        + [pltpu.VMEM((SUB, D), jnp.float32) for _ in range(4)]
        + [pltpu.VMEM((C,), jnp.float32) for _ in range(6)]
        + [pltpu.VMEM((C,), jnp.float32) for _ in range(2)]
        + [pltpu.SemaphoreType.DMA for _ in range(4)]
        + [pltpu.VMEM_SHARED((N_PAD,), jnp.float32) for _ in range(3)]
    )
    f = pl.kernel(
        _sc_body,
        out_type=(
            jax.ShapeDtypeStruct((N_EDGES,), jnp.float32),
            jax.ShapeDtypeStruct((2 * 3 * N_PAD,), jnp.float32),
        ),
        mesh=mesh,
        compiler_params=pltpu.CompilerParams(use_tc_tiling_on_sc=False,
                                             needs_layout_passes=False),
        scratch_types=scratch,
    )
    return f(pos8, idx_i, idx_j, offsets, zeros)


def kernel(positions, idx_i, idx_j, offsets):
    pos8 = jnp.pad(positions, ((0, 0), (0, D - 3)))
    ii = idx_i.astype(jnp.int32)
    ij = idx_j.astype(jnp.int32)
    zeros = jnp.zeros((NR_PER_SUB,), jnp.float32)
    dist, gradp = _sc_call(pos8, ii, ij, offsets.reshape(-1), zeros)
    gp = gradp.reshape(2, 3, N_PAD)
    grad = (gp[0] + gp[1])[:, :N_NODES].T
    return (dist, grad)


# speedup vs baseline: 49.7607x; 12.6345x over previous
import jax
import jax.numpy as jnp
from jax import lax
from jax.experimental import pallas as pl
from jax.experimental.pallas import tpu as pltpu
from jax.experimental.pallas import tpu_sc as plsc

N_NODES = 100000
N_EDGES = 3200000

LANES = 16
D = 8
C = 2000
SUB = 400
NSUBC = C // SUB
E_PER_TILE = N_EDGES // 32
NCH = E_PER_TILE // C
UNROLL = 10
IDX_RING = 5
NSUB = 16
NR_PER_SUB = 6256
N_PAD = NSUB * NR_PER_SUB


def _rsqrt(s):
    bits = plsc.bitcast(s, jnp.int32)
    y = plsc.bitcast(0x5F3759DF - (bits >> 1), jnp.float32)
    half_s = 0.5 * s
    for _ in range(3):
        y = y * (1.5 - half_s * y * y)
    return y


def _sc_body(pos_hbm, idxi_hbm, idxj_hbm, offs_hbm, zeros_hbm,
             dist_hbm, gradp_hbm, *scratch):
    idxi_r = scratch[0:IDX_RING]
    idxj_r = scratch[IDX_RING:2 * IDX_RING]
    k = 2 * IDX_RING
    offs_v = scratch[k]
    posi_s = scratch[k + 1:k + 3]
    posj_s = scratch[k + 3:k + 5]
    u_v = scratch[k + 5:k + 11]
    dist_r = scratch[k + 11:k + 13]
    gsem = scratch[k + 13:k + 15]
    osem = scratch[k + 15]
    ssem = scratch[k + 16]
    g_sh = scratch[k + 17:k + 20]

    cid = lax.axis_index("c")
    sid = lax.axis_index("s")
    wid = sid * 2 + cid
    span = wid * E_PER_TILE

    rows0 = sid * NR_PER_SUB
    for g in g_sh:
        pltpu.sync_copy(zeros_hbm, g.at[pl.ds(rows0, NR_PER_SUB)])
    plsc.subcore_barrier()

    iota16 = lax.iota(jnp.int32, LANES)
    cols = [jnp.full((LANES,), kk, jnp.int32) for kk in range(3)]

    def sub_descs(ring, s, spar):
        ii = idxi_r[ring].at[pl.ds(s * SUB, SUB)]
        jj = idxj_r[ring].at[pl.ds(s * SUB, SUB)]
        return (pltpu.make_async_copy(pos_hbm.at[ii], posi_s[spar],
                                      gsem[spar]),
                pltpu.make_async_copy(pos_hbm.at[jj], posj_s[spar],
                                      gsem[spar]))

    def fire_sub(ring, s, spar):
        a, b = sub_descs(ring, s, spar)
        a.start()
        b.start()

    def wait_sub(ring, s, spar):
        a, b = sub_descs(ring, s, spar)
        a.wait()
        b.wait()

    def scat_descs(ring, par, e0):
        descs = []
        for comp in range(3):
            descs.append(pltpu.make_async_copy(
                u_v[comp], g_sh[comp].at[idxj_r[ring]], ssem))
            descs.append(pltpu.make_async_copy(
                u_v[3 + comp], g_sh[comp].at[idxi_r[ring]], ssem))
        descs.append(pltpu.make_async_copy(
            dist_r[par], dist_hbm.at[pl.ds(e0, C)], ssem))
        return descs

    pltpu.sync_copy(idxi_hbm.at[pl.ds(span, C)], idxi_r[0])
    pltpu.sync_copy(idxj_hbm.at[pl.ds(span, C)], idxj_r[0])
    for comp in range(3):
        pltpu.sync_copy(offs_hbm.at[pl.ds(comp * N_EDGES + span, C)],
                        offs_v.at[pl.ds(comp * C, C)])
    fire_sub(0, 0, 0)

    def compute_sub(s, spar, par):
        posi_v, posj_v = posi_s[spar], posj_s[spar]
        dist_v = dist_r[par]
        base = s * SUB

        def grp(i, carry):
            lridx = i * LANES + iota16
            cbase = base + i * LANES
            cridx = cbase + iota16
            pxi = plsc.load_gather(posi_v, [lridx, cols[0]])
            pyi = plsc.load_gather(posi_v, [lridx, cols[1]])
            pzi = plsc.load_gather(posi_v, [lridx, cols[2]])
            pxj = plsc.load_gather(posj_v, [lridx, cols[0]])
            pyj = plsc.load_gather(posj_v, [lridx, cols[1]])
            pzj = plsc.load_gather(posj_v, [lridx, cols[2]])
            ox = offs_v[pl.ds(cbase, LANES)]
            oy = offs_v[pl.ds(C + cbase, LANES)]
            oz = offs_v[pl.ds(2 * C + cbase, LANES)]
            rx = pxj - pxi + ox
            ry = pyj - pyi + oy
            rz = pzj - pzi + oz
            s2 = rx * rx + ry * ry + rz * rz
            y = _rsqrt(s2)
            dist_v[pl.ds(cbase, LANES)] = s2 * y
            ux, uy, uz = rx * y, ry * y, rz * y
            u_v[0][pl.ds(cbase, LANES)] = ux
            u_v[1][pl.ds(cbase, LANES)] = uy
            u_v[2][pl.ds(cbase, LANES)] = uz
            u_v[3][pl.ds(cbase, LANES)] = -ux
            u_v[4][pl.ds(cbase, LANES)] = -uy
            u_v[5][pl.ds(cbase, LANES)] = -uz
            return carry

        lax.fori_loop(0, SUB // LANES, grp, 0)

    def slot(t, u):
        ring = u % IDX_RING
        ring_n = (u + 1) % IDX_RING
        ring_p = (u - 1) % IDX_RING
        par = u % 2
        par_p = (u - 1) % 2
        e0 = span + t * C

        @pl.when(t > 0)
        def _():
            for comp in range(3):
                pltpu.make_async_copy(
                    offs_hbm.at[pl.ds(comp * N_EDGES + e0, C)],
                    offs_v.at[pl.ds(comp * C, C)], osem).wait()

        wait_sub(ring, 0, 0)

        for s in range(NSUBC):
            if s + 1 < NSUBC:
                fire_sub(ring, s + 1, (s + 1) % 2)
            if s > 0:
                wait_sub(ring, s, s % 2)
            compute_sub(s, s % 2, par)

        for comp in range(3):
            pltpu.sync_copy(u_v[comp], g_sh[comp].at[idxj_r[ring]], add=True)
            pltpu.sync_copy(u_v[3 + comp], g_sh[comp].at[idxi_r[ring]],
                            add=True)
        pltpu.sync_copy(dist_r[par], dist_hbm.at[pl.ds(e0, C)])

        @pl.when(t + 1 < NCH)
        def _():
            en = e0 + C
            pltpu.sync_copy(idxi_hbm.at[pl.ds(en, C)], idxi_r[ring_n])
            pltpu.sync_copy(idxj_hbm.at[pl.ds(en, C)], idxj_r[ring_n])
            for comp in range(3):
                pltpu.async_copy(
                    offs_hbm.at[pl.ds(comp * N_EDGES + en, C)],
                    offs_v.at[pl.ds(comp * C, C)], osem)
            fire_sub(ring_n, 0, 0)

    def outer(kk, carry):
        for u in range(UNROLL):
            slot(kk * UNROLL + u, u)
        return carry

    lax.fori_loop(0, NCH // UNROLL, outer, 0)

    plsc.subcore_barrier()
    for comp in range(3):
        off = (cid * 3 + comp) * N_PAD + rows0
        pltpu.sync_copy(g_sh[comp].at[pl.ds(rows0, NR_PER_SUB)],
                        gradp_hbm.at[pl.ds(off, NR_PER_SUB)])


@jax.jit
def _sc_call(pos8, idx_i, idx_j, offsets, zeros):
    mesh = plsc.VectorSubcoreMesh(core_axis_name="c", subcore_axis_name="s")
    scratch = (
        [pltpu.VMEM((C,), jnp.int32) for _ in range(2 * IDX_RING)]
        + [pltpu.VMEM((3 * C,), jnp.float32)]
        + [pltpu.VMEM((SUB, D), jnp.float32) for _ in range(4)]
        + [pltpu.VMEM((C,), jnp.float32) for _ in range(6)]
        + [pltpu.VMEM((C,), jnp.float32) for _ in range(2)]
        + [pltpu.SemaphoreType.DMA for _ in range(4)]
        + [pltpu.VMEM_SHARED((N_PAD,), jnp.float32) for _ in range(3)]
    )
    f = pl.kernel(
        _sc_body,
        out_type=(
            jax.ShapeDtypeStruct((N_EDGES,), jnp.float32),
            jax.ShapeDtypeStruct((2 * 3 * N_PAD,), jnp.float32),
        ),
        mesh=mesh,
        compiler_params=pltpu.CompilerParams(use_tc_tiling_on_sc=False,
                                             needs_layout_passes=False),
        scratch_types=scratch,
    )
    return f(pos8, idx_i, idx_j, offsets, zeros)


def kernel(positions, idx_i, idx_j, offsets):
    pos8 = jnp.pad(positions, ((0, 0), (0, D - 3)))
    ii = idx_i.astype(jnp.int32)
    ij = idx_j.astype(jnp.int32)
    zeros = jnp.zeros((NR_PER_SUB,), jnp.float32)
    offs_soa = jnp.concatenate([offsets[:, 0], offsets[:, 1], offsets[:, 2]])
    dist, gradp = _sc_call(pos8, ii, ij, offs_soa, zeros)
    gp = gradp.reshape(2, 3, N_PAD)
    grad = (gp[0] + gp[1])[:, :N_NODES].T
    return (dist, grad)


# speedup vs baseline: 61.0856x; 1.2276x over previous
import jax
import jax.numpy as jnp
from jax import lax
from jax.experimental import pallas as pl
from jax.experimental.pallas import tpu as pltpu
from jax.experimental.pallas import tpu_sc as plsc

N_NODES = 100000
N_EDGES = 3200000

LANES = 16
D = 8
C = 2000
SUB = 400
NSUBC = C // SUB
E_PER_TILE = N_EDGES // 32
NCH = E_PER_TILE // C
UNROLL = 10
IDX_RING = 5
NSUB = 16
NR_PER_SUB = 6256
N_PAD = NSUB * NR_PER_SUB


def _rsqrt(s):
    bits = plsc.bitcast(s, jnp.int32)
    y = plsc.bitcast(0x5F3759DF - (bits >> 1), jnp.float32)
    half_s = 0.5 * s
    for _ in range(3):
        y = y * (1.5 - half_s * y * y)
    return y


def _sc_body(pos_hbm, idxi_hbm, idxj_hbm, offs_hbm, zeros_hbm,
             dist_hbm, gradp_hbm, *scratch):
    idxi_r = scratch[0:IDX_RING]
    idxj_r = scratch[IDX_RING:2 * IDX_RING]
    k = 2 * IDX_RING
    offs_v = scratch[k]
    posi_s = scratch[k + 1:k + 3]
    posj_s = scratch[k + 3:k + 5]
    u_v = scratch[k + 5:k + 11]
    dist_r = scratch[k + 11:k + 13]
    gsem = scratch[k + 13:k + 15]
    osem = scratch[k + 15]
    ssem = scratch[k + 16]
    dsem = scratch[k + 17]
    g_sh = scratch[k + 18:k + 21]

    cid = lax.axis_index("c")
    sid = lax.axis_index("s")
    wid = sid * 2 + cid
    span = wid * E_PER_TILE

    rows0 = sid * NR_PER_SUB
    for g in g_sh:
        pltpu.sync_copy(zeros_hbm, g.at[pl.ds(rows0, NR_PER_SUB)])
    plsc.subcore_barrier()

    iota16 = lax.iota(jnp.int32, LANES)
    cols = [jnp.full((LANES,), kk, jnp.int32) for kk in range(3)]

    def sub_descs(ring, s, spar):
        ii = idxi_r[ring].at[pl.ds(s * SUB, SUB)]
        jj = idxj_r[ring].at[pl.ds(s * SUB, SUB)]
        return (pltpu.make_async_copy(pos_hbm.at[ii], posi_s[spar],
                                      gsem[spar]),
                pltpu.make_async_copy(pos_hbm.at[jj], posj_s[spar],
                                      gsem[spar]))

    def fire_sub(ring, s, spar):
        a, b = sub_descs(ring, s, spar)
        a.start()
        b.start()

    def wait_sub(ring, s, spar):
        a, b = sub_descs(ring, s, spar)
        a.wait()
        b.wait()

    def scat_descs(ring, par, e0):
        descs = []
        for comp in range(3):
            descs.append(pltpu.make_async_copy(
                u_v[comp], g_sh[comp].at[idxj_r[ring]], ssem))
            descs.append(pltpu.make_async_copy(
                u_v[3 + comp], g_sh[comp].at[idxi_r[ring]], ssem))
        descs.append(pltpu.make_async_copy(
            dist_r[par], dist_hbm.at[pl.ds(e0, C)], dsem))
        return descs

    pltpu.sync_copy(idxi_hbm.at[pl.ds(span, C)], idxi_r[0])
    pltpu.sync_copy(idxj_hbm.at[pl.ds(span, C)], idxj_r[0])
    for comp in range(3):
        pltpu.sync_copy(offs_hbm.at[pl.ds(comp * N_EDGES + span, C)],
                        offs_v.at[pl.ds(comp * C, C)])
    fire_sub(0, 0, 0)

    def compute_sub(s, spar, par):
        posi_v, posj_v = posi_s[spar], posj_s[spar]
        dist_v = dist_r[par]
        base = s * SUB

        def grp(i, carry):
            lridx = i * LANES + iota16
            cbase = base + i * LANES
            cridx = cbase + iota16
            pxi = plsc.load_gather(posi_v, [lridx, cols[0]])
            pyi = plsc.load_gather(posi_v, [lridx, cols[1]])
            pzi = plsc.load_gather(posi_v, [lridx, cols[2]])
            pxj = plsc.load_gather(posj_v, [lridx, cols[0]])
            pyj = plsc.load_gather(posj_v, [lridx, cols[1]])
            pzj = plsc.load_gather(posj_v, [lridx, cols[2]])
            ox = offs_v[pl.ds(cbase, LANES)]
            oy = offs_v[pl.ds(C + cbase, LANES)]
            oz = offs_v[pl.ds(2 * C + cbase, LANES)]
            rx = pxj - pxi + ox
            ry = pyj - pyi + oy
            rz = pzj - pzi + oz
            s2 = rx * rx + ry * ry + rz * rz
            y = _rsqrt(s2)
            dist_v[pl.ds(cbase, LANES)] = s2 * y
            ux, uy, uz = rx * y, ry * y, rz * y
            u_v[0][pl.ds(cbase, LANES)] = ux
            u_v[1][pl.ds(cbase, LANES)] = uy
            u_v[2][pl.ds(cbase, LANES)] = uz
            u_v[3][pl.ds(cbase, LANES)] = -ux
            u_v[4][pl.ds(cbase, LANES)] = -uy
            u_v[5][pl.ds(cbase, LANES)] = -uz
            return carry

        lax.fori_loop(0, SUB // LANES, grp, 0)

    def slot(t, u):
        ring = u % IDX_RING
        ring_n = (u + 1) % IDX_RING
        ring_p = (u - 1) % IDX_RING
        par = u % 2
        par_p = (u - 1) % 2
        e0 = span + t * C

        @pl.when(t > 0)
        def _():
            for comp in range(3):
                pltpu.make_async_copy(
                    offs_hbm.at[pl.ds(comp * N_EDGES + e0, C)],
                    offs_v.at[pl.ds(comp * C, C)], osem).wait()

        wait_sub(ring, 0, 0)

        @pl.when(t > 0)
        def _():
            for dd in scat_descs(ring_p, par_p, e0 - C):
                dd.wait()

        for s in range(NSUBC):
            if s + 1 < NSUBC:
                fire_sub(ring, s + 1, (s + 1) % 2)
            if s > 0:
                wait_sub(ring, s, s % 2)
            compute_sub(s, s % 2, par)

        descs = scat_descs(ring, par, e0)
        for dd in descs[:6]:
            dd.start(add=True)
        descs[6].start()

        @pl.when(t + 1 < NCH)
        def _():
            en = e0 + C
            pltpu.sync_copy(idxi_hbm.at[pl.ds(en, C)], idxi_r[ring_n])
            pltpu.sync_copy(idxj_hbm.at[pl.ds(en, C)], idxj_r[ring_n])
            for comp in range(3):
                pltpu.async_copy(
                    offs_hbm.at[pl.ds(comp * N_EDGES + en, C)],
                    offs_v.at[pl.ds(comp * C, C)], osem)
            fire_sub(ring_n, 0, 0)

    def outer(kk, carry):
        for u in range(UNROLL):
            slot(kk * UNROLL + u, u)
        return carry

    lax.fori_loop(0, NCH // UNROLL, outer, 0)

    for dd in scat_descs((NCH - 1) % IDX_RING, (NCH - 1) % 2,
                         span + (NCH - 1) * C):
        dd.wait()
    plsc.subcore_barrier()
    for comp in range(3):
        off = (cid * 3 + comp) * N_PAD + rows0
        pltpu.sync_copy(g_sh[comp].at[pl.ds(rows0, NR_PER_SUB)],
                        gradp_hbm.at[pl.ds(off, NR_PER_SUB)])


@jax.jit
def _sc_call(pos8, idx_i, idx_j, offsets, zeros):
    mesh = plsc.VectorSubcoreMesh(core_axis_name="c", subcore_axis_name="s")
    scratch = (
        [pltpu.VMEM((C,), jnp.int32) for _ in range(2 * IDX_RING)]
        + [pltpu.VMEM((3 * C,), jnp.float32)]
        + [pltpu.VMEM((SUB, D), jnp.float32) for _ in range(4)]
        + [pltpu.VMEM((C,), jnp.float32) for _ in range(6)]
        + [pltpu.VMEM((C,), jnp.float32) for _ in range(2)]
        + [pltpu.SemaphoreType.DMA for _ in range(5)]
        + [pltpu.VMEM_SHARED((N_PAD,), jnp.float32) for _ in range(3)]
    )
    f = pl.kernel(
        _sc_body,
        out_type=(
            jax.ShapeDtypeStruct((N_EDGES,), jnp.float32),
            jax.ShapeDtypeStruct((2 * 3 * N_PAD,), jnp.float32),
        ),
        mesh=mesh,
        compiler_params=pltpu.CompilerParams(use_tc_tiling_on_sc=False,
                                             needs_layout_passes=False),
        scratch_types=scratch,
    )
    return f(pos8, idx_i, idx_j, offsets, zeros)


def kernel(positions, idx_i, idx_j, offsets):
    pos8 = jnp.pad(positions, ((0, 0), (0, D - 3)))
    ii = idx_i.astype(jnp.int32)
    ij = idx_j.astype(jnp.int32)
    zeros = jnp.zeros((NR_PER_SUB,), jnp.float32)
    offs_soa = jnp.concatenate([offsets[:, 0], offsets[:, 1], offsets[:, 2]])
    dist, gradp = _sc_call(pos8, ii, ij, offs_soa, zeros)
    gp = gradp.reshape(2, 3, N_PAD)
    grad = (gp[0] + gp[1])[:, :N_NODES].T
    return (dist, grad)
